# Initial kernel scaffold; baseline (speedup 1.0000x reference)
import jax
import jax.numpy as jnp
from jax.experimental import pallas as pl


def kernel(entity_ids, token_embeds, output_entity, edge_index, edge_type, params):
    raise NotImplementedError("write your pallas kernel here")


# speedup vs baseline: 6.1746x; 6.1746x over previous
import functools

import jax
import jax.numpy as jnp
from jax import lax
from jax.experimental import pallas as pl
from jax.experimental.pallas import tpu as pltpu
from jax.experimental.pallas import tpu_sc as plsc

N_ENTITY = 10000
N_EDGES = 160000
NUM_REL = 12
NUM_BASES = 8
HID = 768
ENT_HID = 384
TOK_HID = 768
N_LAYER = 12
N_BLOCK = 2
B = 16
ENT_LEN = 32
TOK_LEN = 128
N_HEAD = 12
HEAD_DIM = 64

NC = 2
NS = 16
LANES = 16

CHUNK = 256
NCHUNK = 40
NPASS = NCHUNK // NC
ACC_ROWS = NUM_REL * CHUNK
ACC_PAD = ACC_ROWS + 128
ZROWS = ACC_PAD // NS
WROWS = ACC_ROWS // NS
EDGE_SHARD = N_EDGES // NS
SUBC = 2000
NSUB = EDGE_SHARD // SUBC
NVREG = SUBC // LANES
RBUF = 32
MROWS = 64

N_PAD = NCHUNK * CHUNK


FW = 128
NFW = ENT_HID // FW


def _seg_accumulate(xs, src, dst, rel):
  mesh = plsc.VectorSubcoreMesh(
      core_axis_name="c", subcore_axis_name="s", num_cores=NC,
      num_subcores=NS)

  z128 = jnp.zeros((ZROWS, FW), jnp.float32)
  z16 = jnp.zeros((ZROWS, 16), jnp.float32)
  ones16 = jnp.ones((RBUF, 16), jnp.float32)

  @functools.partial(
      pl.kernel, mesh=mesh,
      compiler_params=pltpu.CompilerParams(needs_layout_passes=False),
      out_type=(
          [jax.ShapeDtypeStruct((NCHUNK, ACC_ROWS, FW), jnp.float32)
           for _ in range(NFW)],
          jax.ShapeDtypeStruct((NCHUNK, ACC_ROWS, 16), jnp.float32),
      ),
      scratch_types=[
          [pltpu.VMEM_SHARED((ACC_PAD, FW), jnp.float32)
           for _ in range(NFW)],
          pltpu.VMEM_SHARED((ACC_PAD, 16), jnp.float32),
          pltpu.VMEM((SUBC,), jnp.int32),
          pltpu.VMEM((SUBC,), jnp.int32),
          pltpu.VMEM((SUBC,), jnp.int32),
          pltpu.VMEM((MROWS, RBUF), jnp.int32),
          pltpu.VMEM((MROWS, RBUF), jnp.int32),
          [pltpu.VMEM((RBUF, FW), jnp.float32) for _ in range(NFW)],
          pltpu.VMEM((RBUF, 16), jnp.float32),
          pltpu.SemaphoreType.DMA,
      ],
  )
  def k(xa_hbm, xb_hbm, xc_hbm, src_hbm, dst_hbm, rel_hbm, z128_hbm,
        z16_hbm, ones_hbm, s_hbm, cnt_hbm, accs, cacc, srcb, dstb,
        relb, msrc, mkey, rows, ones_v, sem):
    x_hbms = (xa_hbm, xb_hbm, xc_hbm)
    sc = lax.axis_index("c")
    t = lax.axis_index("s")
    grow = ACC_ROWS + t
    pltpu.sync_copy(ones_hbm, ones_v)

    def do_pass(p, carry):
      chunk = NC * p + sc
      base_node = chunk * CHUNK
      for w in range(NFW):
        pltpu.sync_copy(z128_hbm, accs[w].at[pl.ds(t * ZROWS, ZROWS)])
      pltpu.sync_copy(z16_hbm, cacc.at[pl.ds(t * ZROWS, ZROWS)])
      plsc.subcore_barrier()

      def do_sub(g, carry2):
        eoff = t * EDGE_SHARD + g * SUBC
        pltpu.sync_copy(src_hbm.at[pl.ds(eoff, SUBC)], srcb)
        pltpu.sync_copy(dst_hbm.at[pl.ds(eoff, SUBC)], dstb)
        pltpu.sync_copy(rel_hbm.at[pl.ds(eoff, SUBC)], relb)

        def do_vreg(v, m):
          s16 = srcb[pl.ds(v * LANES, LANES)]
          d16 = dstb[pl.ds(v * LANES, LANES)]
          r16 = relb[pl.ds(v * LANES, LANES)]
          loc = d16 - base_node
          inr = (loc >= 0) & (loc < CHUNK)
          key = r16 * CHUNK + loc
          inr_i = inr.astype(jnp.int32)
          pos = m + plsc.cumsum(inr_i) - inr_i
          plsc.store_scatter(msrc, [pos // RBUF, pos % RBUF], s16, mask=inr)
          plsc.store_scatter(mkey, [pos // RBUF, pos % RBUF], key, mask=inr)
          return m + jnp.sum(inr_i)

        m = lax.fori_loop(0, NVREG, do_vreg, 0)
        lanes = lax.iota(jnp.int32, LANES)
        pad_s = jnp.zeros((LANES,), jnp.int32) + t * 8
        pad_k = jnp.zeros((LANES,), jnp.int32) + grow
        full = jnp.ones((LANES,), jnp.bool_)
        for i in range(RBUF // LANES):
          ppos = m + i * LANES + lanes
          plsc.store_scatter(msrc, [ppos // RBUF, ppos % RBUF], pad_s,
                             mask=full)
          plsc.store_scatter(mkey, [ppos // RBUF, ppos % RBUF], pad_k,
                             mask=full)
        nb = (m + RBUF - 1) // RBUF

        def do_batch(j, carry3):
          for w in range(NFW):
            pltpu.async_copy(x_hbms[w].at[msrc.at[j]], rows[w],
                             sem).wait()
            pltpu.sync_copy(rows[w], accs[w].at[mkey.at[j]], add=True)
          pltpu.sync_copy(ones_v, cacc.at[mkey.at[j]], add=True)
          return carry3

        lax.fori_loop(0, nb, do_batch, 0)
        return carry2

      lax.fori_loop(0, NSUB, do_sub, 0)
      plsc.subcore_barrier()
      for w in range(NFW):
        pltpu.sync_copy(accs[w].at[pl.ds(t * WROWS, WROWS)],
                        s_hbm[w].at[chunk, pl.ds(t * WROWS, WROWS)])
      pltpu.sync_copy(cacc.at[pl.ds(t * WROWS, WROWS)],
                      cnt_hbm.at[chunk, pl.ds(t * WROWS, WROWS)])
      plsc.subcore_barrier()
      return carry

    lax.fori_loop(0, NPASS, do_pass, 0)

  return k(xs[0], xs[1], xs[2], src, dst, rel, z128, z16, ones16)


def _entity_select(table, idx):
  nw = NC * NS
  b_per_w = idx.shape[0] // nw
  mesh = plsc.VectorSubcoreMesh(
      core_axis_name="c", subcore_axis_name="s", num_cores=NC,
      num_subcores=NS)

  @functools.partial(
      pl.kernel, mesh=mesh,
      compiler_params=pltpu.CompilerParams(needs_layout_passes=False),
      out_type=jax.ShapeDtypeStruct((idx.shape[0], HID), jnp.float32),
      scratch_types=[
          pltpu.VMEM((b_per_w,), jnp.int32),
          pltpu.VMEM((b_per_w, HID), jnp.float32),
          pltpu.SemaphoreType.DMA,
      ],
  )
  def k(table_hbm, idx_hbm, out_hbm, idx_v, rows_v, sem):
    wid = lax.axis_index("s") * NC + lax.axis_index("c")
    base = wid * b_per_w
    pltpu.sync_copy(idx_hbm.at[pl.ds(base, b_per_w)], idx_v)
    pltpu.async_copy(table_hbm.at[idx_v], rows_v, sem).wait()
    pltpu.sync_copy(rows_v, out_hbm.at[pl.ds(base, b_per_w)])

  return k(table, idx)


def _dot(a, b):
  return jnp.dot(a, b, preferred_element_type=jnp.float32)


def _basis_combine(comp, bases_flat):
  def body(c_ref, b_ref, o_ref):
    o_ref[...] = _dot(c_ref[...], b_ref[...])

  return pl.pallas_call(
      body,
      out_shape=jax.ShapeDtypeStruct((NUM_REL, ENT_HID * ENT_HID),
                                     jnp.float32),
  )(comp, bases_flat)


def _rgcn_block(x_pad, s, cnt, ww, biases):
  rgcn_b, w1, b1, w2, b2, pw, pb = biases

  def body(x_ref, s0_ref, s1_ref, s2_ref, c_ref, w_ref, rb_ref, w1_ref,
           b1_ref, w2_ref, b2_ref, pw_ref, pb_ref, o_ref):
    x = x_ref[...]
    cntc = jnp.clip(c_ref[0, :, 0:1], 1.0, None)
    t_all = jnp.concatenate([s0_ref[0], s1_ref[0], s2_ref[0]],
                            axis=1) / cntc
    h = _dot(x, w_ref[0]) + rb_ref[...] + x
    for r in range(NUM_REL):
      h = h + _dot(t_all[r * CHUNK:(r + 1) * CHUNK], w_ref[r + 1])
    e = jnp.maximum(_dot(h, w1_ref[...]) + b1_ref[...], 0.0)
    e = _dot(e, w2_ref[...]) + b2_ref[...] + h
    o_ref[...] = _dot(e, pw_ref[...]) + pb_ref[...]

  nchunks = NCHUNK
  return pl.pallas_call(
      body,
      grid=(nchunks,),
      in_specs=[
          pl.BlockSpec((CHUNK, ENT_HID), lambda i: (i, 0)),
          pl.BlockSpec((1, ACC_ROWS, FW), lambda i: (i, 0, 0)),
          pl.BlockSpec((1, ACC_ROWS, FW), lambda i: (i, 0, 0)),
          pl.BlockSpec((1, ACC_ROWS, FW), lambda i: (i, 0, 0)),
          pl.BlockSpec((1, ACC_ROWS, 16), lambda i: (i, 0, 0)),
          pl.BlockSpec((NUM_REL + 1, ENT_HID, ENT_HID),
                       lambda i: (0, 0, 0)),
          pl.BlockSpec((1, ENT_HID), lambda i: (0, 0)),
          pl.BlockSpec((ENT_HID, ENT_HID // 2), lambda i: (0, 0)),
          pl.BlockSpec((1, ENT_HID // 2), lambda i: (0, 0)),
          pl.BlockSpec((ENT_HID // 2, ENT_HID), lambda i: (0, 0)),
          pl.BlockSpec((1, ENT_HID), lambda i: (0, 0)),
          pl.BlockSpec((ENT_HID, HID), lambda i: (0, 0)),
          pl.BlockSpec((1, HID), lambda i: (0, 0)),
      ],
      out_specs=pl.BlockSpec((CHUNK, HID), lambda i: (i, 0)),
      out_shape=jax.ShapeDtypeStruct((nchunks * CHUNK, HID), jnp.float32),
  )(x_pad[:nchunks * CHUNK], s[0], s[1], s[2], cnt, ww, rgcn_b, w1, b1,
    w2, b2, pw, pb)


def _token_proj(tok_flat, w1, b1, w2, b2, pw, pb):
  def body(t_ref, w1_ref, b1_ref, w2_ref, b2_ref, pw_ref, pb_ref, o_ref):
    t = t_ref[...]
    h = jnp.maximum(_dot(t, w1_ref[...]) + b1_ref[...], 0.0)
    h = _dot(h, w2_ref[...]) + b2_ref[...] + t
    o_ref[...] = _dot(h, pw_ref[...]) + pb_ref[...]

  n = tok_flat.shape[0]
  blk = 256
  return pl.pallas_call(
      body,
      grid=(n // blk,),
      in_specs=[
          pl.BlockSpec((blk, TOK_HID), lambda i: (i, 0)),
          pl.BlockSpec((TOK_HID, TOK_HID // 2), lambda i: (0, 0)),
          pl.BlockSpec((1, TOK_HID // 2), lambda i: (0, 0)),
          pl.BlockSpec((TOK_HID // 2, TOK_HID), lambda i: (0, 0)),
          pl.BlockSpec((1, TOK_HID), lambda i: (0, 0)),
          pl.BlockSpec((TOK_HID, HID), lambda i: (0, 0)),
          pl.BlockSpec((1, HID), lambda i: (0, 0)),
      ],
      out_specs=pl.BlockSpec((blk, HID), lambda i: (i, 0)),
      out_shape=jax.ShapeDtypeStruct((n, HID), jnp.float32),
  )(tok_flat, w1, b1, w2, b2, pw, pb)


def _cross_prompt(tok3, ent_sel, cross_w, w1, b1, w2, b2):
  def body(t_ref, e_ref, cw_ref, w1_ref, b1_ref, w2_ref, b2_ref, o_ref):
    t = t_ref[0]
    e = e_ref[0]
    q = _dot(t, cw_ref[...])
    scores = lax.dot_general(q, e, (((1,), (1,)), ((), ())),
                             preferred_element_type=jnp.float32) / HID
    mx = jnp.max(scores, axis=0, keepdims=True)
    ex = jnp.exp(scores - mx)
    tw = ex / jnp.sum(ex, axis=0, keepdims=True)
    p = lax.dot_general(tw, t, (((0,), (0,)), ((), ())),
                        preferred_element_type=jnp.float32) + e
    h = jnp.maximum(_dot(p, w1_ref[...]) + b1_ref[...], 0.0)
    o_ref[0] = _dot(h, w2_ref[...]) + b2_ref[...] + p

  return pl.pallas_call(
      body,
      grid=(B,),
      in_specs=[
          pl.BlockSpec((1, TOK_LEN, HID), lambda i: (i, 0, 0)),
          pl.BlockSpec((1, ENT_LEN, HID), lambda i: (i, 0, 0)),
          pl.BlockSpec((HID, HID), lambda i: (0, 0)),
          pl.BlockSpec((HID, HID // 2), lambda i: (0, 0)),
          pl.BlockSpec((1, HID // 2), lambda i: (0, 0)),
          pl.BlockSpec((HID // 2, HID), lambda i: (0, 0)),
          pl.BlockSpec((1, HID), lambda i: (0, 0)),
      ],
      out_specs=pl.BlockSpec((1, ENT_LEN, HID), lambda i: (i, 0, 0)),
      out_shape=jax.ShapeDtypeStruct((B, ENT_LEN, HID), jnp.float32),
  )(tok3, ent_sel, cross_w, w1, b1, w2, b2)


def _prompt_proj(p2, pw, pb):
  nout = N_LAYER * N_BLOCK * HID
  blk = 2048

  def body(p_ref, w_ref, b_ref, o_ref):
    o_ref[...] = _dot(p_ref[...], w_ref[...]) + b_ref[...]

  return pl.pallas_call(
      body,
      grid=(nout // blk,),
      in_specs=[
          pl.BlockSpec((B * ENT_LEN, HID), lambda j: (0, 0)),
          pl.BlockSpec((HID, blk), lambda j: (0, j)),
          pl.BlockSpec((1, blk), lambda j: (0, j)),
      ],
      out_specs=pl.BlockSpec((B * ENT_LEN, blk), lambda j: (0, j)),
      out_shape=jax.ShapeDtypeStruct((B * ENT_LEN, nout), jnp.float32),
  )(p2, pw, pb)


def kernel(entity_ids, token_embeds, output_entity, edge_index, edge_type,
           params):
  p = params
  x = p['node_embeds'].astype(jnp.float32)
  src = edge_index[0].astype(jnp.int32)
  dst = edge_index[1].astype(jnp.int32)
  rel = edge_type.astype(jnp.int32)

  xs = [x[:, w * FW:(w + 1) * FW] for w in range(NFW)]
  s, cnt = _seg_accumulate(xs, src, dst, rel)

  w_flat = _basis_combine(p['rgcn_comp'],
                          p['rgcn_bases'].reshape(NUM_BASES,
                                                  ENT_HID * ENT_HID))
  w = w_flat.reshape(NUM_REL, ENT_HID, ENT_HID)
  ww = jnp.concatenate([p['rgcn_root'][None], w], axis=0)

  x_pad = jnp.concatenate(
      [x, jnp.zeros((N_PAD - N_ENTITY, ENT_HID), jnp.float32)], axis=0)
  biases = (p['rgcn_bias'][None], p['ep1w1'], p['ep1b1'][None],
            p['ep1w2'], p['ep1b2'][None], p['ep2w'], p['ep2b'][None])
  ent2 = _rgcn_block(x_pad, s, cnt, ww, biases)

  tok_flat = token_embeds.reshape(B * TOK_LEN, TOK_HID)
  tok3 = _token_proj(tok_flat, p['tp1w1'], p['tp1b1'][None], p['tp1w2'],
                     p['tp1b2'][None], p['tp2w'], p['tp2b'][None])
  tok3 = tok3.reshape(B, TOK_LEN, HID)

  ent_sel = _entity_select(ent2, entity_ids.reshape(-1).astype(jnp.int32))
  ent_sel = ent_sel.reshape(B, ENT_LEN, HID)

  p2 = _cross_prompt(tok3, ent_sel, p['cross_w'], p['pp1w1'],
                     p['pp1b1'][None], p['pp1w2'], p['pp1b2'][None])

  out = _prompt_proj(p2.reshape(B * ENT_LEN, HID), p['pp2w'],
                     p['pp2b'][None])

  prompt = out.reshape(B, ENT_LEN, N_LAYER, N_BLOCK, N_HEAD, HEAD_DIM)
  return jnp.transpose(prompt, (2, 3, 0, 4, 1, 5))


# speedup vs baseline: 8.1441x; 1.3190x over previous
import functools

import jax
import jax.numpy as jnp
from jax import lax
from jax.experimental import pallas as pl
from jax.experimental.pallas import tpu as pltpu
from jax.experimental.pallas import tpu_sc as plsc

N_ENTITY = 10000
N_EDGES = 160000
NUM_REL = 12
NUM_BASES = 8
HID = 768
ENT_HID = 384
TOK_HID = 768
N_LAYER = 12
N_BLOCK = 2
B = 16
ENT_LEN = 32
TOK_LEN = 128
N_HEAD = 12
HEAD_DIM = 64

NC = 2
NS = 16
LANES = 16

CHUNK = 256
NCHUNK = 40
NPASS = NCHUNK // NC
ACC_ROWS = NUM_REL * CHUNK
ACC_PAD = ACC_ROWS + 128
ZROWS = ACC_PAD // NS
WROWS = ACC_ROWS // NS
EDGE_SHARD = N_EDGES // NS
SUBC = 2000
NSUB = EDGE_SHARD // SUBC
NVREG = SUBC // LANES
RBUF = 32
MROWS = 64

N_PAD = NCHUNK * CHUNK


FW = 128
NFW = ENT_HID // FW


def _seg_accumulate(xs, edges_packed):
  mesh = plsc.VectorSubcoreMesh(
      core_axis_name="c", subcore_axis_name="s", num_cores=NC,
      num_subcores=NS)

  z128 = jnp.zeros((ZROWS, FW), jnp.float32)
  z16 = jnp.zeros((ZROWS, 16), jnp.float32)
  ones16 = jnp.ones((RBUF, 16), jnp.float32)

  @functools.partial(
      pl.kernel, mesh=mesh,
      compiler_params=pltpu.CompilerParams(needs_layout_passes=False),
      out_type=(
          [jax.ShapeDtypeStruct((NCHUNK, ACC_ROWS, FW), jnp.float32)
           for _ in range(NFW)],
          jax.ShapeDtypeStruct((NCHUNK, ACC_ROWS, 16), jnp.float32),
      ),
      scratch_types=[
          [pltpu.VMEM_SHARED((ACC_PAD, FW), jnp.float32)
           for _ in range(NFW)],
          pltpu.VMEM_SHARED((ACC_PAD, 16), jnp.float32),
          pltpu.VMEM((3 * SUBC,), jnp.int32),
          pltpu.VMEM((MROWS, RBUF), jnp.int32),
          pltpu.VMEM((MROWS, RBUF), jnp.int32),
          [pltpu.VMEM((RBUF, FW), jnp.float32) for _ in range(NFW)],
          pltpu.VMEM((RBUF, 16), jnp.float32),
          pltpu.SemaphoreType.DMA,
      ],
  )
  def k(xa_hbm, xb_hbm, xc_hbm, e_hbm, z128_hbm, z16_hbm, ones_hbm,
        s_hbm, cnt_hbm, accs, cacc, edgeb, msrc, mkey, rows, ones_v,
        sem):
    x_hbms = (xa_hbm, xb_hbm, xc_hbm)
    sc = lax.axis_index("c")
    t = lax.axis_index("s")
    grow = ACC_ROWS + t
    pltpu.sync_copy(ones_hbm, ones_v)

    def do_pass(p, carry):
      chunk = NC * p + sc
      base_node = chunk * CHUNK
      for w in range(NFW):
        pltpu.sync_copy(z128_hbm, accs[w].at[pl.ds(t * ZROWS, ZROWS)])
      pltpu.sync_copy(z16_hbm, cacc.at[pl.ds(t * ZROWS, ZROWS)])
      plsc.subcore_barrier()

      def do_sub(g, carry2):
        eoff = (t * NSUB + g) * 3 * SUBC
        pltpu.sync_copy(e_hbm.at[pl.ds(eoff, 3 * SUBC)], edgeb)

        def do_vreg(v, m):
          s16 = edgeb[pl.ds(v * LANES, LANES)]
          d16 = edgeb[pl.ds(SUBC + v * LANES, LANES)]
          r16 = edgeb[pl.ds(2 * SUBC + v * LANES, LANES)]
          loc = d16 - base_node
          inr = (loc >= 0) & (loc < CHUNK)
          key = r16 * CHUNK + loc
          inr_i = inr.astype(jnp.int32)
          pos = m + plsc.cumsum(inr_i) - inr_i
          plsc.store_scatter(msrc, [pos // RBUF, pos % RBUF], s16, mask=inr)
          plsc.store_scatter(mkey, [pos // RBUF, pos % RBUF], key, mask=inr)
          return m + jnp.sum(inr_i)

        m = lax.fori_loop(0, NVREG, do_vreg, 0)
        lanes = lax.iota(jnp.int32, LANES)
        pad_s = jnp.zeros((LANES,), jnp.int32) + t * 8
        pad_k = jnp.zeros((LANES,), jnp.int32) + grow
        full = jnp.ones((LANES,), jnp.bool_)
        for i in range(RBUF // LANES):
          ppos = m + i * LANES + lanes
          plsc.store_scatter(msrc, [ppos // RBUF, ppos % RBUF], pad_s,
                             mask=full)
          plsc.store_scatter(mkey, [ppos // RBUF, ppos % RBUF], pad_k,
                             mask=full)
        nb = (m + RBUF - 1) // RBUF

        def do_batch(j, carry3):
          handles = [pltpu.async_copy(x_hbms[w].at[msrc.at[j]], rows[w],
                                      sem) for w in range(NFW)]
          pltpu.sync_copy(ones_v, cacc.at[mkey.at[j]], add=True)
          for h in handles:
            h.wait()
          for w in range(NFW):
            pltpu.sync_copy(rows[w], accs[w].at[mkey.at[j]], add=True)
          return carry3

        lax.fori_loop(0, nb, do_batch, 0)
        return carry2

      lax.fori_loop(0, NSUB, do_sub, 0)
      plsc.subcore_barrier()
      for w in range(NFW):
        pltpu.sync_copy(accs[w].at[pl.ds(t * WROWS, WROWS)],
                        s_hbm[w].at[chunk, pl.ds(t * WROWS, WROWS)])
      pltpu.sync_copy(cacc.at[pl.ds(t * WROWS, WROWS)],
                      cnt_hbm.at[chunk, pl.ds(t * WROWS, WROWS)])
      plsc.subcore_barrier()
      return carry

    lax.fori_loop(0, NPASS, do_pass, 0)

  return k(xs[0], xs[1], xs[2], edges_packed, z128, z16, ones16)


def _entity_select(table, idx):
  nw = NC * NS
  b_per_w = idx.shape[0] // nw
  mesh = plsc.VectorSubcoreMesh(
      core_axis_name="c", subcore_axis_name="s", num_cores=NC,
      num_subcores=NS)

  @functools.partial(
      pl.kernel, mesh=mesh,
      compiler_params=pltpu.CompilerParams(needs_layout_passes=False),
      out_type=jax.ShapeDtypeStruct((idx.shape[0], HID), jnp.float32),
      scratch_types=[
          pltpu.VMEM((b_per_w,), jnp.int32),
          pltpu.VMEM((b_per_w, HID), jnp.float32),
          pltpu.SemaphoreType.DMA,
      ],
  )
  def k(table_hbm, idx_hbm, out_hbm, idx_v, rows_v, sem):
    wid = lax.axis_index("s") * NC + lax.axis_index("c")
    base = wid * b_per_w
    pltpu.sync_copy(idx_hbm.at[pl.ds(base, b_per_w)], idx_v)
    pltpu.async_copy(table_hbm.at[idx_v], rows_v, sem).wait()
    pltpu.sync_copy(rows_v, out_hbm.at[pl.ds(base, b_per_w)])

  return k(table, idx)


def _dot(a, b):
  return jnp.dot(a, b, preferred_element_type=jnp.float32)


def _basis_combine(comp, bases_flat):
  def body(c_ref, b_ref, o_ref):
    o_ref[...] = _dot(c_ref[...], b_ref[...])

  return pl.pallas_call(
      body,
      out_shape=jax.ShapeDtypeStruct((NUM_REL, ENT_HID * ENT_HID),
                                     jnp.float32),
  )(comp, bases_flat)


def _rgcn_block(x_pad, s, cnt, ww, biases):
  rgcn_b, w1, b1, w2, b2, pw, pb = biases

  def body(x_ref, s0_ref, s1_ref, s2_ref, c_ref, w_ref, rb_ref, w1_ref,
           b1_ref, w2_ref, b2_ref, pw_ref, pb_ref, o_ref):
    x = x_ref[...]
    cntc = jnp.clip(c_ref[0, :, 0:1], 1.0, None)
    t_all = jnp.concatenate([s0_ref[0], s1_ref[0], s2_ref[0]],
                            axis=1) / cntc
    h = _dot(x, w_ref[0]) + rb_ref[...] + x
    for r in range(NUM_REL):
      h = h + _dot(t_all[r * CHUNK:(r + 1) * CHUNK], w_ref[r + 1])
    e = jnp.maximum(_dot(h, w1_ref[...]) + b1_ref[...], 0.0)
    e = _dot(e, w2_ref[...]) + b2_ref[...] + h
    o_ref[...] = _dot(e, pw_ref[...]) + pb_ref[...]

  nchunks = NCHUNK
  return pl.pallas_call(
      body,
      grid=(nchunks,),
      in_specs=[
          pl.BlockSpec((CHUNK, ENT_HID), lambda i: (i, 0)),
          pl.BlockSpec((1, ACC_ROWS, FW), lambda i: (i, 0, 0)),
          pl.BlockSpec((1, ACC_ROWS, FW), lambda i: (i, 0, 0)),
          pl.BlockSpec((1, ACC_ROWS, FW), lambda i: (i, 0, 0)),
          pl.BlockSpec((1, ACC_ROWS, 16), lambda i: (i, 0, 0)),
          pl.BlockSpec((NUM_REL + 1, ENT_HID, ENT_HID),
                       lambda i: (0, 0, 0)),
          pl.BlockSpec((1, ENT_HID), lambda i: (0, 0)),
          pl.BlockSpec((ENT_HID, ENT_HID // 2), lambda i: (0, 0)),
          pl.BlockSpec((1, ENT_HID // 2), lambda i: (0, 0)),
          pl.BlockSpec((ENT_HID // 2, ENT_HID), lambda i: (0, 0)),
          pl.BlockSpec((1, ENT_HID), lambda i: (0, 0)),
          pl.BlockSpec((ENT_HID, HID), lambda i: (0, 0)),
          pl.BlockSpec((1, HID), lambda i: (0, 0)),
      ],
      out_specs=pl.BlockSpec((CHUNK, HID), lambda i: (i, 0)),
      out_shape=jax.ShapeDtypeStruct((nchunks * CHUNK, HID), jnp.float32),
  )(x_pad[:nchunks * CHUNK], s[0], s[1], s[2], cnt, ww, rgcn_b, w1, b1,
    w2, b2, pw, pb)


def _token_proj(tok_flat, w1, b1, w2, b2, pw, pb):
  def body(t_ref, w1_ref, b1_ref, w2_ref, b2_ref, pw_ref, pb_ref, o_ref):
    t = t_ref[...]
    h = jnp.maximum(_dot(t, w1_ref[...]) + b1_ref[...], 0.0)
    h = _dot(h, w2_ref[...]) + b2_ref[...] + t
    o_ref[...] = _dot(h, pw_ref[...]) + pb_ref[...]

  n = tok_flat.shape[0]
  blk = 256
  return pl.pallas_call(
      body,
      grid=(n // blk,),
      in_specs=[
          pl.BlockSpec((blk, TOK_HID), lambda i: (i, 0)),
          pl.BlockSpec((TOK_HID, TOK_HID // 2), lambda i: (0, 0)),
          pl.BlockSpec((1, TOK_HID // 2), lambda i: (0, 0)),
          pl.BlockSpec((TOK_HID // 2, TOK_HID), lambda i: (0, 0)),
          pl.BlockSpec((1, TOK_HID), lambda i: (0, 0)),
          pl.BlockSpec((TOK_HID, HID), lambda i: (0, 0)),
          pl.BlockSpec((1, HID), lambda i: (0, 0)),
      ],
      out_specs=pl.BlockSpec((blk, HID), lambda i: (i, 0)),
      out_shape=jax.ShapeDtypeStruct((n, HID), jnp.float32),
  )(tok_flat, w1, b1, w2, b2, pw, pb)


def _cross_prompt(tok3, ent_sel, cross_w, w1, b1, w2, b2):
  def body(t_ref, e_ref, cw_ref, w1_ref, b1_ref, w2_ref, b2_ref, o_ref):
    t = t_ref[0]
    e = e_ref[0]
    q = _dot(t, cw_ref[...])
    scores = lax.dot_general(q, e, (((1,), (1,)), ((), ())),
                             preferred_element_type=jnp.float32) / HID
    mx = jnp.max(scores, axis=0, keepdims=True)
    ex = jnp.exp(scores - mx)
    tw = ex / jnp.sum(ex, axis=0, keepdims=True)
    p = lax.dot_general(tw, t, (((0,), (0,)), ((), ())),
                        preferred_element_type=jnp.float32) + e
    h = jnp.maximum(_dot(p, w1_ref[...]) + b1_ref[...], 0.0)
    o_ref[0] = _dot(h, w2_ref[...]) + b2_ref[...] + p

  return pl.pallas_call(
      body,
      grid=(B,),
      in_specs=[
          pl.BlockSpec((1, TOK_LEN, HID), lambda i: (i, 0, 0)),
          pl.BlockSpec((1, ENT_LEN, HID), lambda i: (i, 0, 0)),
          pl.BlockSpec((HID, HID), lambda i: (0, 0)),
          pl.BlockSpec((HID, HID // 2), lambda i: (0, 0)),
          pl.BlockSpec((1, HID // 2), lambda i: (0, 0)),
          pl.BlockSpec((HID // 2, HID), lambda i: (0, 0)),
          pl.BlockSpec((1, HID), lambda i: (0, 0)),
      ],
      out_specs=pl.BlockSpec((1, ENT_LEN, HID), lambda i: (i, 0, 0)),
      out_shape=jax.ShapeDtypeStruct((B, ENT_LEN, HID), jnp.float32),
  )(tok3, ent_sel, cross_w, w1, b1, w2, b2)


def _prompt_proj(p2, pw, pb):
  nout = N_LAYER * N_BLOCK * HID
  blk = 2048

  def body(p_ref, w_ref, b_ref, o_ref):
    o_ref[...] = _dot(p_ref[...], w_ref[...]) + b_ref[...]

  return pl.pallas_call(
      body,
      grid=(nout // blk,),
      in_specs=[
          pl.BlockSpec((B * ENT_LEN, HID), lambda j: (0, 0)),
          pl.BlockSpec((HID, blk), lambda j: (0, j)),
          pl.BlockSpec((1, blk), lambda j: (0, j)),
      ],
      out_specs=pl.BlockSpec((B * ENT_LEN, blk), lambda j: (0, j)),
      out_shape=jax.ShapeDtypeStruct((B * ENT_LEN, nout), jnp.float32),
  )(p2, pw, pb)


def kernel(entity_ids, token_embeds, output_entity, edge_index, edge_type,
           params):
  p = params
  x = p['node_embeds'].astype(jnp.float32)
  src = edge_index[0].astype(jnp.int32)
  dst = edge_index[1].astype(jnp.int32)
  rel = edge_type.astype(jnp.int32)

  edges_packed = jnp.stack(
      [src.reshape(NS, NSUB, SUBC), dst.reshape(NS, NSUB, SUBC),
       rel.reshape(NS, NSUB, SUBC)], axis=2).reshape(-1)
  xs = [x[:, w * FW:(w + 1) * FW] for w in range(NFW)]
  s, cnt = _seg_accumulate(xs, edges_packed)

  w_flat = _basis_combine(p['rgcn_comp'],
                          p['rgcn_bases'].reshape(NUM_BASES,
                                                  ENT_HID * ENT_HID))
  w = w_flat.reshape(NUM_REL, ENT_HID, ENT_HID)
  ww = jnp.concatenate([p['rgcn_root'][None], w], axis=0)

  x_pad = jnp.concatenate(
      [x, jnp.zeros((N_PAD - N_ENTITY, ENT_HID), jnp.float32)], axis=0)
  biases = (p['rgcn_bias'][None], p['ep1w1'], p['ep1b1'][None],
            p['ep1w2'], p['ep1b2'][None], p['ep2w'], p['ep2b'][None])
  ent2 = _rgcn_block(x_pad, s, cnt, ww, biases)

  tok_flat = token_embeds.reshape(B * TOK_LEN, TOK_HID)
  tok3 = _token_proj(tok_flat, p['tp1w1'], p['tp1b1'][None], p['tp1w2'],
                     p['tp1b2'][None], p['tp2w'], p['tp2b'][None])
  tok3 = tok3.reshape(B, TOK_LEN, HID)

  ent_sel = _entity_select(ent2, entity_ids.reshape(-1).astype(jnp.int32))
  ent_sel = ent_sel.reshape(B, ENT_LEN, HID)

  p2 = _cross_prompt(tok3, ent_sel, p['cross_w'], p['pp1w1'],
                     p['pp1b1'][None], p['pp1w2'], p['pp1b2'][None])

  out = _prompt_proj(p2.reshape(B * ENT_LEN, HID), p['pp2w'],
                     p['pp2b'][None])

  prompt = out.reshape(B, ENT_LEN, N_LAYER, N_BLOCK, N_HEAD, HEAD_DIM)
  return jnp.transpose(prompt, (2, 3, 0, 4, 1, 5))
